# baseline (speedup 1.0000x reference)
import functools

import jax
import jax.numpy as jnp
from jax import lax
from jax.experimental import pallas as pl
from jax.experimental.pallas import tpu as pltpu
from jax.experimental.pallas import tpu_sc as plsc

BATCH = 16384
D = 64
NC, NS = 2, 16
NW = NC * NS
B_PER_W = BATCH // NW
HALF = 256
TCOLS = 2048


def _pair_pack(tT):
    n = tT.shape[1]
    nblk = pl.cdiv(n, TCOLS)
    grid = (nblk,)

    def body(x_ref, o_ref):
        x = x_ref[...]
        rows = lax.broadcasted_iota(jnp.int32, (D, D), 0)
        cols = lax.broadcasted_iota(jnp.int32, (D, D), 1)
        ident = (rows == cols).astype(jnp.float32)
        dn = (((0,), (0,)), ((), ()))
        t0 = lax.dot_general(x[:, :TCOLS // 2], ident, dn,
                             preferred_element_type=jnp.float32)
        t1 = lax.dot_general(x[:, TCOLS // 2:], ident, dn,
                             preferred_element_type=jnp.float32)
        o_ref[...] = jnp.concatenate([t0, t1], axis=1)

    return pl.pallas_call(
        body,
        grid=grid,
        in_specs=[pl.BlockSpec((D, TCOLS), lambda i: (0, i))],
        out_specs=pl.BlockSpec((TCOLS // 2, 2 * D), lambda i: (i, 0)),
        out_shape=jax.ShapeDtypeStruct((nblk * (TCOLS // 2), 2 * D),
                                       jnp.float32),
        compiler_params=pltpu.CompilerParams(
            fuse_transposed_lhs_in_matmul=True),
    )(tT)


def _gather_concat(user_ids, movie_ids, utp, mtp):
    mesh = plsc.VectorSubcoreMesh(core_axis_name="c", subcore_axis_name="s")

    @functools.partial(
        pl.kernel,
        mesh=mesh,
        compiler_params=pltpu.CompilerParams(needs_layout_passes=False),
        out_type=jax.ShapeDtypeStruct((BATCH, 2 * D), jnp.float32),
        scratch_types=[
            pltpu.VMEM_SHARED((NS, B_PER_W), jnp.int32),
            pltpu.SMEM((2 * B_PER_W,), jnp.int32),
            pltpu.VMEM((HALF, 2 * D), jnp.float32),
            pltpu.VMEM((B_PER_W, 2 * D), jnp.float32),
            pltpu.SemaphoreType.DMA,
        ],
    )
    def gather_k(uid_hbm, mid_hbm, ut_hbm, mt_hbm, comb_hbm, sh, sm, pair,
                 comb, sem):
        s = lax.axis_index("s")
        wid = s * NC + lax.axis_index("c")
        base = wid * B_PER_W
        pltpu.sync_copy(uid_hbm.at[pl.ds(base, B_PER_W)], sh.at[s])
        pltpu.sync_copy(sh.at[s], sm.at[pl.ds(0, B_PER_W)])
        pltpu.sync_copy(mid_hbm.at[pl.ds(base, B_PER_W)], sh.at[s])
        pltpu.sync_copy(sh.at[s], sm.at[pl.ds(B_PER_W, B_PER_W)])

        def phase(t_hbm, id_off, coff, half):
            def issue(kk, _):
                idx = sm[id_off + half * HALF + kk]
                blk = lax.shift_right_logical(idx, 11)
                r = lax.bitwise_and(idx, TCOLS - 1)
                prow = blk * (TCOLS // 2) + lax.bitwise_and(r, TCOLS // 2 - 1)
                pltpu.async_copy(t_hbm.at[prow], pair.at[kk], sem)
                return 0
            lax.fori_loop(0, HALF, issue, 0, unroll=4)
            def drain(kk, _):
                pltpu.make_async_copy(t_hbm.at[0], pair.at[kk], sem).wait()
                return 0
            lax.fori_loop(0, HALF, drain, 0, unroll=4)
            def extract(kk, _):
                idx = sm[id_off + half * HALF + kk]
                off = lax.shift_right_logical(
                    lax.bitwise_and(idx, TCOLS - 1), 10) * D
                row = half * HALF + kk
                for g in range(D // 16):
                    comb[row, pl.ds(coff + g * 16, 16)] = (
                        pair[kk, pl.ds(off + g * 16, 16)])
                return 0
            lax.fori_loop(0, HALF, extract, 0, unroll=2)

        for half in range(B_PER_W // HALF):
            phase(ut_hbm, 0, 0, half)
        for half in range(B_PER_W // HALF):
            phase(mt_hbm, B_PER_W, D, half)
        pltpu.sync_copy(comb, comb_hbm.at[pl.ds(base, B_PER_W)])

    return gather_k(user_ids, movie_ids, utp, mtp)


def _mlp_block(x_ref, w1_ref, b1_ref, w2_ref, b2_ref, w3_ref, b3_ref, o_ref):
    h = (jnp.dot(x_ref[...], w1_ref[...], preferred_element_type=jnp.float32)
         + b1_ref[...])
    h = jnp.maximum(h, 0.0)
    h = jnp.dot(h, w2_ref[...], preferred_element_type=jnp.float32) + b2_ref[...]
    h = jnp.maximum(h, 0.0)
    r = jnp.dot(h, w3_ref[...], preferred_element_type=jnp.float32) + b3_ref[...]
    o_ref[...] = jax.nn.sigmoid(r)


def _mlp(comb, W1, b1, W2, b2, W3, b3):
    NB = 2048
    grid = (BATCH // NB,)

    def full(shape):
        return pl.BlockSpec(shape, lambda i: (0,) * len(shape))

    return pl.pallas_call(
        _mlp_block,
        grid=grid,
        in_specs=[
            pl.BlockSpec((NB, 2 * D), lambda i: (i, 0)),
            full((2 * D, 128)),
            full((1, 128)),
            full((128, D)),
            full((1, D)),
            full((D, 1)),
            full((1, 1)),
        ],
        out_specs=pl.BlockSpec((NB, 1), lambda i: (i, 0)),
        out_shape=jax.ShapeDtypeStruct((BATCH, 1), jnp.float32),
    )(comb, W1, b1.reshape(1, 128), W2, b2.reshape(1, D), W3,
      b3.reshape(1, 1))


def kernel(user_ids, movie_ids, user_table, movie_table,
           W1, b1, W2, b2, W3, b3):
    utp = _pair_pack(user_table.T)
    mtp = _pair_pack(movie_table.T)
    comb = _gather_concat(user_ids.astype(jnp.int32),
                          movie_ids.astype(jnp.int32), utp, mtp)
    rating = _mlp(comb, W1, b1, W2, b2, W3, b3)
    return rating.reshape(BATCH)


# speedup vs baseline: 1.6319x; 1.6319x over previous
import functools

import jax
import jax.numpy as jnp
from jax import lax
from jax.experimental import pallas as pl
from jax.experimental.pallas import tpu as pltpu
from jax.experimental.pallas import tpu_sc as plsc

BATCH = 16384
D = 64
NC, NS = 2, 16
NW = NC * NS
B_PER_W = BATCH // NW
HALF = 256
TCOLS = 8192


def _pair_pack(tT):
    n = tT.shape[1]
    nblk = pl.cdiv(n, TCOLS)
    grid = (nblk,)

    def body(x_ref, o_ref):
        x = x_ref[...]
        rows = lax.broadcasted_iota(jnp.int32, (D, D), 0)
        cols = lax.broadcasted_iota(jnp.int32, (D, D), 1)
        ident = (rows == cols).astype(jnp.float32)
        dn = (((0,), (0,)), ((), ()))
        t0 = lax.dot_general(x[:, :TCOLS // 2], ident, dn,
                             preferred_element_type=jnp.float32)
        t1 = lax.dot_general(x[:, TCOLS // 2:], ident, dn,
                             preferred_element_type=jnp.float32)
        o_ref[...] = jnp.concatenate([t0, t1], axis=1)

    return pl.pallas_call(
        body,
        grid=grid,
        in_specs=[pl.BlockSpec((D, TCOLS), lambda i: (0, i))],
        out_specs=pl.BlockSpec((TCOLS // 2, 2 * D), lambda i: (i, 0)),
        out_shape=jax.ShapeDtypeStruct((nblk * (TCOLS // 2), 2 * D),
                                       jnp.float32),
        compiler_params=pltpu.CompilerParams(
            fuse_transposed_lhs_in_matmul=True),
    )(tT)


def _gather_concat(user_ids, movie_ids, utp, mtp):
    mesh = plsc.VectorSubcoreMesh(core_axis_name="c", subcore_axis_name="s")

    @functools.partial(
        pl.kernel,
        mesh=mesh,
        compiler_params=pltpu.CompilerParams(needs_layout_passes=False),
        out_type=jax.ShapeDtypeStruct((BATCH, 2 * D), jnp.float32),
        scratch_types=[
            pltpu.VMEM_SHARED((NS, B_PER_W), jnp.int32),
            pltpu.SMEM((2 * B_PER_W,), jnp.int32),
            pltpu.VMEM((HALF, 2 * D), jnp.float32),
            pltpu.VMEM((B_PER_W, 2 * D), jnp.float32),
            pltpu.SemaphoreType.DMA,
        ],
    )
    def gather_k(uid_hbm, mid_hbm, ut_hbm, mt_hbm, comb_hbm, sh, sm, pair,
                 comb, sem):
        s = lax.axis_index("s")
        wid = s * NC + lax.axis_index("c")
        base = wid * B_PER_W
        pltpu.sync_copy(uid_hbm.at[pl.ds(base, B_PER_W)], sh.at[s])
        pltpu.sync_copy(sh.at[s], sm.at[pl.ds(0, B_PER_W)])
        pltpu.sync_copy(mid_hbm.at[pl.ds(base, B_PER_W)], sh.at[s])
        pltpu.sync_copy(sh.at[s], sm.at[pl.ds(B_PER_W, B_PER_W)])

        def phase(t_hbm, id_off, coff, half):
            def issue(kk, _):
                idx = sm[id_off + half * HALF + kk]
                blk = lax.shift_right_logical(idx, 13)
                r = lax.bitwise_and(idx, TCOLS - 1)
                prow = blk * (TCOLS // 2) + lax.bitwise_and(r, TCOLS // 2 - 1)
                pltpu.async_copy(t_hbm.at[prow], pair.at[kk], sem)
                return 0
            lax.fori_loop(0, HALF, issue, 0, unroll=4)
            def drain(kk, _):
                pltpu.make_async_copy(t_hbm.at[0], pair.at[kk], sem).wait()
                return 0
            lax.fori_loop(0, HALF, drain, 0, unroll=4)
            def extract(kk, _):
                idx = sm[id_off + half * HALF + kk]
                off = lax.shift_right_logical(
                    lax.bitwise_and(idx, TCOLS - 1), 12) * D
                row = half * HALF + kk
                for g in range(D // 16):
                    comb[row, pl.ds(coff + g * 16, 16)] = (
                        pair[kk, pl.ds(off + g * 16, 16)])
                return 0
            lax.fori_loop(0, HALF, extract, 0, unroll=2)

        for half in range(B_PER_W // HALF):
            phase(ut_hbm, 0, 0, half)
        for half in range(B_PER_W // HALF):
            phase(mt_hbm, B_PER_W, D, half)
        pltpu.sync_copy(comb, comb_hbm.at[pl.ds(base, B_PER_W)])

    return gather_k(user_ids, movie_ids, utp, mtp)


def _mlp_block(x_ref, w1_ref, b1_ref, w2_ref, b2_ref, w3_ref, b3_ref, o_ref):
    h = (jnp.dot(x_ref[...], w1_ref[...], preferred_element_type=jnp.float32)
         + b1_ref[...])
    h = jnp.maximum(h, 0.0)
    h = jnp.dot(h, w2_ref[...], preferred_element_type=jnp.float32) + b2_ref[...]
    h = jnp.maximum(h, 0.0)
    r = jnp.dot(h, w3_ref[...], preferred_element_type=jnp.float32) + b3_ref[...]
    o_ref[...] = jax.nn.sigmoid(r)


def _mlp(comb, W1, b1, W2, b2, W3, b3):
    NB = 2048
    grid = (BATCH // NB,)

    def full(shape):
        return pl.BlockSpec(shape, lambda i: (0,) * len(shape))

    return pl.pallas_call(
        _mlp_block,
        grid=grid,
        in_specs=[
            pl.BlockSpec((NB, 2 * D), lambda i: (i, 0)),
            full((2 * D, 128)),
            full((1, 128)),
            full((128, D)),
            full((1, D)),
            full((D, 1)),
            full((1, 1)),
        ],
        out_specs=pl.BlockSpec((NB, 1), lambda i: (i, 0)),
        out_shape=jax.ShapeDtypeStruct((BATCH, 1), jnp.float32),
    )(comb, W1, b1.reshape(1, 128), W2, b2.reshape(1, D), W3,
      b3.reshape(1, 1))


def kernel(user_ids, movie_ids, user_table, movie_table,
           W1, b1, W2, b2, W3, b3):
    utp = _pair_pack(user_table.T)
    mtp = _pair_pack(movie_table.T)
    comb = _gather_concat(user_ids.astype(jnp.int32),
                          movie_ids.astype(jnp.int32), utp, mtp)
    rating = _mlp(comb, W1, b1, W2, b2, W3, b3)
    return rating.reshape(BATCH)


# speedup vs baseline: 1.8012x; 1.1038x over previous
import functools

import jax
import jax.numpy as jnp
from jax import lax
from jax.experimental import pallas as pl
from jax.experimental.pallas import tpu as pltpu
from jax.experimental.pallas import tpu_sc as plsc

BATCH = 16384
D = 64
NC, NS = 2, 16
NW = NC * NS
B_PER_W = BATCH // NW
HALF = 256
TCOLS = 16384
TSHIFT = 14


def _pair_pack(tT):
    n = tT.shape[1]
    nblk = pl.cdiv(n, TCOLS)
    grid = (nblk,)

    def body(x_ref, o_ref):
        x = x_ref[...]
        rows = lax.broadcasted_iota(jnp.int32, (D, D), 0)
        cols = lax.broadcasted_iota(jnp.int32, (D, D), 1)
        ident = (rows == cols).astype(jnp.float32)
        dn = (((0,), (0,)), ((), ()))
        t0 = lax.dot_general(x[:, :TCOLS // 2], ident, dn,
                             preferred_element_type=jnp.float32)
        t1 = lax.dot_general(x[:, TCOLS // 2:], ident, dn,
                             preferred_element_type=jnp.float32)
        o_ref[...] = jnp.concatenate([t0, t1], axis=1)

    return pl.pallas_call(
        body,
        grid=grid,
        in_specs=[pl.BlockSpec((D, TCOLS), lambda i: (0, i))],
        out_specs=pl.BlockSpec((TCOLS // 2, 2 * D), lambda i: (i, 0)),
        out_shape=jax.ShapeDtypeStruct((nblk * (TCOLS // 2), 2 * D),
                                       jnp.float32),
        compiler_params=pltpu.CompilerParams(
            fuse_transposed_lhs_in_matmul=True),
    )(tT)


def _gather_concat(user_ids, movie_ids, utp, mtp):
    mesh = plsc.VectorSubcoreMesh(core_axis_name="c", subcore_axis_name="s")

    @functools.partial(
        pl.kernel,
        mesh=mesh,
        compiler_params=pltpu.CompilerParams(needs_layout_passes=False),
        out_type=jax.ShapeDtypeStruct((BATCH, 2 * D), jnp.float32),
        scratch_types=[
            pltpu.VMEM_SHARED((NS, B_PER_W), jnp.int32),
            pltpu.SMEM((2 * B_PER_W,), jnp.int32),
            pltpu.VMEM((HALF, 2 * D), jnp.float32),
            pltpu.VMEM((B_PER_W, 2 * D), jnp.float32),
            pltpu.SemaphoreType.DMA,
        ],
    )
    def gather_k(uid_hbm, mid_hbm, ut_hbm, mt_hbm, comb_hbm, sh, sm, pair,
                 comb, sem):
        s = lax.axis_index("s")
        wid = s * NC + lax.axis_index("c")
        base = wid * B_PER_W
        pltpu.sync_copy(uid_hbm.at[pl.ds(base, B_PER_W)], sh.at[s])
        pltpu.sync_copy(sh.at[s], sm.at[pl.ds(0, B_PER_W)])
        pltpu.sync_copy(mid_hbm.at[pl.ds(base, B_PER_W)], sh.at[s])
        pltpu.sync_copy(sh.at[s], sm.at[pl.ds(B_PER_W, B_PER_W)])

        def phase(t_hbm, id_off, coff, half):
            def issue(kk, _):
                idx = sm[id_off + half * HALF + kk]
                blk = lax.shift_right_logical(idx, TSHIFT)
                r = lax.bitwise_and(idx, TCOLS - 1)
                prow = blk * (TCOLS // 2) + lax.bitwise_and(r, TCOLS // 2 - 1)
                pltpu.async_copy(t_hbm.at[prow], pair.at[kk], sem)
                return 0
            lax.fori_loop(0, HALF, issue, 0, unroll=4)
            def drain(kk, _):
                pltpu.make_async_copy(t_hbm.at[0], pair.at[kk], sem).wait()
                return 0
            lax.fori_loop(0, HALF, drain, 0, unroll=4)
            def extract(kk, _):
                idx = sm[id_off + half * HALF + kk]
                off = lax.shift_right_logical(
                    lax.bitwise_and(idx, TCOLS - 1), TSHIFT - 1) * D
                row = half * HALF + kk
                for g in range(D // 16):
                    comb[row, pl.ds(coff + g * 16, 16)] = (
                        pair[kk, pl.ds(off + g * 16, 16)])
                return 0
            lax.fori_loop(0, HALF, extract, 0, unroll=2)

        for half in range(B_PER_W // HALF):
            phase(ut_hbm, 0, 0, half)
        for half in range(B_PER_W // HALF):
            phase(mt_hbm, B_PER_W, D, half)
        pltpu.sync_copy(comb, comb_hbm.at[pl.ds(base, B_PER_W)])

    return gather_k(user_ids, movie_ids, utp, mtp)


def _mlp_block(x_ref, w1_ref, b1_ref, w2_ref, b2_ref, w3_ref, b3_ref, o_ref):
    h = (jnp.dot(x_ref[...], w1_ref[...], preferred_element_type=jnp.float32)
         + b1_ref[...])
    h = jnp.maximum(h, 0.0)
    h = jnp.dot(h, w2_ref[...], preferred_element_type=jnp.float32) + b2_ref[...]
    h = jnp.maximum(h, 0.0)
    r = jnp.dot(h, w3_ref[...], preferred_element_type=jnp.float32) + b3_ref[...]
    o_ref[...] = jax.nn.sigmoid(r)


def _mlp(comb, W1, b1, W2, b2, W3, b3):
    NB = 2048
    grid = (BATCH // NB,)

    def full(shape):
        return pl.BlockSpec(shape, lambda i: (0,) * len(shape))

    return pl.pallas_call(
        _mlp_block,
        grid=grid,
        in_specs=[
            pl.BlockSpec((NB, 2 * D), lambda i: (i, 0)),
            full((2 * D, 128)),
            full((1, 128)),
            full((128, D)),
            full((1, D)),
            full((D, 1)),
            full((1, 1)),
        ],
        out_specs=pl.BlockSpec((NB, 1), lambda i: (i, 0)),
        out_shape=jax.ShapeDtypeStruct((BATCH, 1), jnp.float32),
    )(comb, W1, b1.reshape(1, 128), W2, b2.reshape(1, D), W3,
      b3.reshape(1, 1))


def kernel(user_ids, movie_ids, user_table, movie_table,
           W1, b1, W2, b2, W3, b3):
    utp = _pair_pack(user_table.T)
    mtp = _pair_pack(movie_table.T)
    comb = _gather_concat(user_ids.astype(jnp.int32),
                          movie_ids.astype(jnp.int32), utp, mtp)
    rating = _mlp(comb, W1, b1, W2, b2, W3, b3)
    return rating.reshape(BATCH)


# speedup vs baseline: 1.8664x; 1.0362x over previous
import functools

import jax
import jax.numpy as jnp
from jax import lax
from jax.experimental import pallas as pl
from jax.experimental.pallas import tpu as pltpu
from jax.experimental.pallas import tpu_sc as plsc

BATCH = 16384
D = 64
NC, NS = 2, 16
NW = NC * NS
B_PER_W = BATCH // NW
HALF = 256
TCOLS = 32768
TSHIFT = 15


def _pair_pack(tT):
    n = tT.shape[1]
    nblk = pl.cdiv(n, TCOLS)
    grid = (nblk,)

    def body(x_ref, o_ref):
        x = x_ref[...]
        rows = lax.broadcasted_iota(jnp.int32, (D, D), 0)
        cols = lax.broadcasted_iota(jnp.int32, (D, D), 1)
        ident = (rows == cols).astype(jnp.float32)
        dn = (((0,), (0,)), ((), ()))
        t0 = lax.dot_general(x[:, :TCOLS // 2], ident, dn,
                             preferred_element_type=jnp.float32)
        t1 = lax.dot_general(x[:, TCOLS // 2:], ident, dn,
                             preferred_element_type=jnp.float32)
        o_ref[...] = jnp.concatenate([t0, t1], axis=1)

    return pl.pallas_call(
        body,
        grid=grid,
        in_specs=[pl.BlockSpec((D, TCOLS), lambda i: (0, i))],
        out_specs=pl.BlockSpec((TCOLS // 2, 2 * D), lambda i: (i, 0)),
        out_shape=jax.ShapeDtypeStruct((nblk * (TCOLS // 2), 2 * D),
                                       jnp.float32),
        compiler_params=pltpu.CompilerParams(
            fuse_transposed_lhs_in_matmul=True),
    )(tT)


def _gather_concat(user_ids, movie_ids, utp, mtp):
    mesh = plsc.VectorSubcoreMesh(core_axis_name="c", subcore_axis_name="s")

    @functools.partial(
        pl.kernel,
        mesh=mesh,
        compiler_params=pltpu.CompilerParams(needs_layout_passes=False),
        out_type=jax.ShapeDtypeStruct((BATCH, 2 * D), jnp.float32),
        scratch_types=[
            pltpu.VMEM_SHARED((NS, B_PER_W), jnp.int32),
            pltpu.SMEM((2 * B_PER_W,), jnp.int32),
            pltpu.VMEM((HALF, 2 * D), jnp.float32),
            pltpu.VMEM((B_PER_W, 2 * D), jnp.float32),
            pltpu.SemaphoreType.DMA,
        ],
    )
    def gather_k(uid_hbm, mid_hbm, ut_hbm, mt_hbm, comb_hbm, sh, sm, pair,
                 comb, sem):
        s = lax.axis_index("s")
        wid = s * NC + lax.axis_index("c")
        base = wid * B_PER_W
        pltpu.sync_copy(uid_hbm.at[pl.ds(base, B_PER_W)], sh.at[s])
        pltpu.sync_copy(sh.at[s], sm.at[pl.ds(0, B_PER_W)])
        pltpu.sync_copy(mid_hbm.at[pl.ds(base, B_PER_W)], sh.at[s])
        pltpu.sync_copy(sh.at[s], sm.at[pl.ds(B_PER_W, B_PER_W)])

        def phase(t_hbm, id_off, coff, half):
            def issue(kk, _):
                idx = sm[id_off + half * HALF + kk]
                blk = lax.shift_right_logical(idx, TSHIFT)
                r = lax.bitwise_and(idx, TCOLS - 1)
                prow = blk * (TCOLS // 2) + lax.bitwise_and(r, TCOLS // 2 - 1)
                pltpu.async_copy(t_hbm.at[prow], pair.at[kk], sem)
                return 0
            lax.fori_loop(0, HALF, issue, 0, unroll=4)
            def drain(kk, _):
                pltpu.make_async_copy(t_hbm.at[0], pair.at[kk], sem).wait()
                return 0
            lax.fori_loop(0, HALF, drain, 0, unroll=4)
            def extract(kk, _):
                idx = sm[id_off + half * HALF + kk]
                off = lax.shift_right_logical(
                    lax.bitwise_and(idx, TCOLS - 1), TSHIFT - 1) * D
                row = half * HALF + kk
                for g in range(D // 16):
                    comb[row, pl.ds(coff + g * 16, 16)] = (
                        pair[kk, pl.ds(off + g * 16, 16)])
                return 0
            lax.fori_loop(0, HALF, extract, 0, unroll=2)

        for half in range(B_PER_W // HALF):
            phase(ut_hbm, 0, 0, half)
        for half in range(B_PER_W // HALF):
            phase(mt_hbm, B_PER_W, D, half)
        pltpu.sync_copy(comb, comb_hbm.at[pl.ds(base, B_PER_W)])

    return gather_k(user_ids, movie_ids, utp, mtp)


def _mlp_block(x_ref, w1_ref, b1_ref, w2_ref, b2_ref, w3_ref, b3_ref, o_ref):
    h = (jnp.dot(x_ref[...], w1_ref[...], preferred_element_type=jnp.float32)
         + b1_ref[...])
    h = jnp.maximum(h, 0.0)
    h = jnp.dot(h, w2_ref[...], preferred_element_type=jnp.float32) + b2_ref[...]
    h = jnp.maximum(h, 0.0)
    r = jnp.dot(h, w3_ref[...], preferred_element_type=jnp.float32) + b3_ref[...]
    o_ref[...] = jax.nn.sigmoid(r)


def _mlp(comb, W1, b1, W2, b2, W3, b3):
    NB = 2048
    grid = (BATCH // NB,)

    def full(shape):
        return pl.BlockSpec(shape, lambda i: (0,) * len(shape))

    return pl.pallas_call(
        _mlp_block,
        grid=grid,
        in_specs=[
            pl.BlockSpec((NB, 2 * D), lambda i: (i, 0)),
            full((2 * D, 128)),
            full((1, 128)),
            full((128, D)),
            full((1, D)),
            full((D, 1)),
            full((1, 1)),
        ],
        out_specs=pl.BlockSpec((NB, 1), lambda i: (i, 0)),
        out_shape=jax.ShapeDtypeStruct((BATCH, 1), jnp.float32),
    )(comb, W1, b1.reshape(1, 128), W2, b2.reshape(1, D), W3,
      b3.reshape(1, 1))


def kernel(user_ids, movie_ids, user_table, movie_table,
           W1, b1, W2, b2, W3, b3):
    utp = _pair_pack(user_table.T)
    mtp = _pair_pack(movie_table.T)
    comb = _gather_concat(user_ids.astype(jnp.int32),
                          movie_ids.astype(jnp.int32), utp, mtp)
    rating = _mlp(comb, W1, b1, W2, b2, W3, b3)
    return rating.reshape(BATCH)


# speedup vs baseline: 1.8999x; 1.0179x over previous
import functools

import jax
import jax.numpy as jnp
from jax import lax
from jax.experimental import pallas as pl
from jax.experimental.pallas import tpu as pltpu
from jax.experimental.pallas import tpu_sc as plsc

BATCH = 16384
D = 64
NC, NS = 2, 16
NW = NC * NS
B_PER_W = BATCH // NW
HALF = 256
TCOLS = 32768
TSHIFT = 15
MCOLS = 8192
MSHIFT = 13


def _pair_pack(tT, tcols):
    n = tT.shape[1]
    nblk = pl.cdiv(n, tcols)
    grid = (nblk,)

    def body(x_ref, o_ref):
        x = x_ref[...]
        rows = lax.broadcasted_iota(jnp.int32, (D, D), 0)
        cols = lax.broadcasted_iota(jnp.int32, (D, D), 1)
        ident = (rows == cols).astype(jnp.float32)
        dn = (((0,), (0,)), ((), ()))
        t0 = lax.dot_general(x[:, :tcols // 2], ident, dn,
                             preferred_element_type=jnp.float32)
        t1 = lax.dot_general(x[:, tcols // 2:], ident, dn,
                             preferred_element_type=jnp.float32)
        o_ref[...] = jnp.concatenate([t0, t1], axis=1)

    return pl.pallas_call(
        body,
        grid=grid,
        in_specs=[pl.BlockSpec((D, tcols), lambda i: (0, i))],
        out_specs=pl.BlockSpec((tcols // 2, 2 * D), lambda i: (i, 0)),
        out_shape=jax.ShapeDtypeStruct((nblk * (tcols // 2), 2 * D),
                                       jnp.float32),
        compiler_params=pltpu.CompilerParams(
            fuse_transposed_lhs_in_matmul=True),
    )(tT)


def _gather_concat(user_ids, movie_ids, utp, mtp):
    mesh = plsc.VectorSubcoreMesh(core_axis_name="c", subcore_axis_name="s")

    @functools.partial(
        pl.kernel,
        mesh=mesh,
        compiler_params=pltpu.CompilerParams(needs_layout_passes=False),
        out_type=jax.ShapeDtypeStruct((BATCH, 2 * D), jnp.float32),
        scratch_types=[
            pltpu.VMEM_SHARED((NS, B_PER_W), jnp.int32),
            pltpu.SMEM((2 * B_PER_W,), jnp.int32),
            pltpu.VMEM((HALF, 2 * D), jnp.float32),
            pltpu.VMEM((B_PER_W, 2 * D), jnp.float32),
            pltpu.SemaphoreType.DMA,
        ],
    )
    def gather_k(uid_hbm, mid_hbm, ut_hbm, mt_hbm, comb_hbm, sh, sm, pair,
                 comb, sem):
        s = lax.axis_index("s")
        wid = s * NC + lax.axis_index("c")
        base = wid * B_PER_W
        pltpu.sync_copy(uid_hbm.at[pl.ds(base, B_PER_W)], sh.at[s])
        pltpu.sync_copy(sh.at[s], sm.at[pl.ds(0, B_PER_W)])
        pltpu.sync_copy(mid_hbm.at[pl.ds(base, B_PER_W)], sh.at[s])
        pltpu.sync_copy(sh.at[s], sm.at[pl.ds(B_PER_W, B_PER_W)])

        def phase(t_hbm, id_off, coff, half, tsh, tcols):
            def issue(kk, _):
                idx = sm[id_off + half * HALF + kk]
                blk = lax.shift_right_logical(idx, tsh)
                r = lax.bitwise_and(idx, tcols - 1)
                prow = blk * (tcols // 2) + lax.bitwise_and(r, tcols // 2 - 1)
                pltpu.async_copy(t_hbm.at[prow], pair.at[kk], sem)
                return 0
            lax.fori_loop(0, HALF, issue, 0, unroll=4)
            def drain(kk, _):
                pltpu.make_async_copy(t_hbm.at[0], pair.at[kk], sem).wait()
                return 0
            lax.fori_loop(0, HALF, drain, 0, unroll=4)
            def extract(kk, _):
                idx = sm[id_off + half * HALF + kk]
                off = lax.shift_right_logical(
                    lax.bitwise_and(idx, tcols - 1), tsh - 1) * D
                row = half * HALF + kk
                for g in range(D // 16):
                    comb[row, pl.ds(coff + g * 16, 16)] = (
                        pair[kk, pl.ds(off + g * 16, 16)])
                return 0
            lax.fori_loop(0, HALF, extract, 0, unroll=2)

        for half in range(B_PER_W // HALF):
            phase(ut_hbm, 0, 0, half, TSHIFT, TCOLS)
        for half in range(B_PER_W // HALF):
            phase(mt_hbm, B_PER_W, D, half, MSHIFT, MCOLS)
        pltpu.sync_copy(comb, comb_hbm.at[pl.ds(base, B_PER_W)])

    return gather_k(user_ids, movie_ids, utp, mtp)


def _mlp_block(x_ref, w1_ref, b1_ref, w2_ref, b2_ref, w3_ref, b3_ref, o_ref):
    h = (jnp.dot(x_ref[...], w1_ref[...], preferred_element_type=jnp.float32)
         + b1_ref[...])
    h = jnp.maximum(h, 0.0)
    h = jnp.dot(h, w2_ref[...], preferred_element_type=jnp.float32) + b2_ref[...]
    h = jnp.maximum(h, 0.0)
    r = jnp.dot(h, w3_ref[...], preferred_element_type=jnp.float32) + b3_ref[...]
    o_ref[...] = jax.nn.sigmoid(r)[:, 0]


def _mlp(comb, W1, b1, W2, b2, W3, b3):
    NB = 2048
    grid = (BATCH // NB,)

    def full(shape):
        return pl.BlockSpec(shape, lambda i: (0,) * len(shape))

    return pl.pallas_call(
        _mlp_block,
        grid=grid,
        in_specs=[
            pl.BlockSpec((NB, 2 * D), lambda i: (i, 0)),
            full((2 * D, 128)),
            full((1, 128)),
            full((128, D)),
            full((1, D)),
            full((D, 1)),
            full((1, 1)),
        ],
        out_specs=pl.BlockSpec((NB,), lambda i: (i,)),
        out_shape=jax.ShapeDtypeStruct((BATCH,), jnp.float32),
    )(comb, W1, b1.reshape(1, 128), W2, b2.reshape(1, D), W3,
      b3.reshape(1, 1))


def kernel(user_ids, movie_ids, user_table, movie_table,
           W1, b1, W2, b2, W3, b3):
    utp = _pair_pack(user_table.T, TCOLS)
    mtp = _pair_pack(movie_table.T, MCOLS)
    comb = _gather_concat(user_ids.astype(jnp.int32),
                          movie_ids.astype(jnp.int32), utp, mtp)
    return _mlp(comb, W1, b1, W2, b2, W3, b3)
